# baseline (speedup 1.0000x reference)
import functools

import jax
import jax.numpy as jnp
from jax import lax
from jax.experimental import pallas as pl
from jax.experimental.pallas import tpu as pltpu
from jax.experimental.pallas import tpu_sc as plsc

_TAU = 10.0
_LANES = 16


def _make_sc_fn(B, L, S):
    f32, i32 = jnp.float32, jnp.int32
    NC = 2
    mesh = plsc.VectorSubcoreMesh(
        core_axis_name="c", subcore_axis_name="s", num_cores=NC, num_subcores=16)

    @functools.partial(
        pl.kernel,
        mesh=mesh,
        compiler_params=pltpu.CompilerParams(needs_layout_passes=False),
        out_type=[
            jax.ShapeDtypeStruct((B, S), f32),
            jax.ShapeDtypeStruct((B, S), f32),
            jax.ShapeDtypeStruct((B, S), i32),
            jax.ShapeDtypeStruct((B, S), i32),
        ],
        scratch_types=[
            pltpu.VMEM((L,), f32),
            pltpu.VMEM((L,), i32),
            pltpu.VMEM((S,), f32),
            pltpu.VMEM((2 * _LANES,), f32),
            pltpu.VMEM((S,), f32),
            pltpu.VMEM((S,), f32),
            pltpu.VMEM((S,), i32),
            pltpu.VMEM((S,), i32),
        ],
    )
    def sc_fn(time_hbm, type_hbm, u_hbm, fs_hbm,
              tlo_hbm, dlt_hbm, g_hbm, acc_hbm,
              t_v, ty_v, u_v, fs_v, tlo_v, dlt_v, g_v, acc_v):
        wid = lax.axis_index("s") * NC + lax.axis_index("c")
        pltpu.sync_copy(time_hbm.at[wid], t_v)
        pltpu.sync_copy(type_hbm.at[wid], ty_v)
        pltpu.sync_copy(u_hbm.at[wid], u_v)
        pltpu.sync_copy(fs_hbm.at[wid], fs_v)

        first = fs_v[pl.ds(0, _LANES)]
        scale = fs_v[pl.ds(_LANES, _LANES)]

        def search(base):
            st = u_v[pl.ds(base, _LANES)] * scale + first
            sthi = st + _TAU
            pos = jnp.zeros((_LANES,), i32)
            pos2 = jnp.zeros((_LANES,), i32)
            step = L // 2
            while step >= 1:
                cand = pos + step
                cand2 = pos2 + step
                tc = plsc.load_gather(t_v, [cand])
                tc2 = plsc.load_gather(t_v, [cand2])
                pos = jnp.where(tc <= st, cand, pos)
                pos2 = jnp.where(tc2 <= sthi, cand2, pos2)
                step //= 2
            t_lo = plsc.load_gather(t_v, [pos])
            g16 = plsc.load_gather(ty_v, [pos])
            return st, pos, pos2, t_lo, g16

        def scan_store(base, srch):
            st, pos, pos2, t_lo, g16 = srch

            w = pos2 - pos

            def wbody(state):
                j, acc = state
                for r in range(4):
                    jr = j + r
                    idx = jnp.minimum(pos + 1 + jr, L - 1)
                    tyj = plsc.load_gather(ty_v, [idx])
                    bit = jnp.where(jr < w, jnp.left_shift(jnp.int32(1), tyj), 0)
                    acc = acc | bit
                return (j + jnp.int32(4), acc)

            _, acc = lax.while_loop(
                lambda s: jnp.any(s[0] < w), wbody,
                (jnp.int32(0), jnp.zeros((_LANES,), i32)))

            tlo_v[pl.ds(base, _LANES)] = t_lo
            dlt_v[pl.ds(base, _LANES)] = st - t_lo
            g_v[pl.ds(base, _LANES)] = g16
            acc_v[pl.ds(base, _LANES)] = acc

        def chunk(i, carry):
            base_a = i * 2 * _LANES
            base_b = base_a + _LANES
            sa = search(base_a)
            sb = search(base_b)
            scan_store(base_a, sa)
            scan_store(base_b, sb)
            return carry

        lax.fori_loop(0, S // (2 * _LANES), chunk, jnp.int32(0))

        pltpu.sync_copy(tlo_v, tlo_hbm.at[wid])
        pltpu.sync_copy(dlt_v, dlt_hbm.at[wid])
        pltpu.sync_copy(g_v, g_hbm.at[wid])
        pltpu.sync_copy(acc_v, acc_hbm.at[wid])

    return sc_fn


def _tc_body(tlo_ref, dlt_ref, g_ref, acc_ref, emb_ref, wt_ref, w1_ref,
             w1dt_ref, b1_ref, w2_ref, b2_ref, probs_ref, label_ref):
    B, S = tlo_ref.shape
    K, D = emb_ref.shape
    f32, i32 = jnp.float32, jnp.int32
    hi = lax.Precision.HIGHEST
    dn0 = (((0,), (0,)), ((), ()))

    kk_k = lax.broadcasted_iota(i32, (K, S), 0)
    eye_k = (lax.broadcasted_iota(i32, (K, K), 0)
             == lax.broadcasted_iota(i32, (K, K), 1)).astype(f32)
    w1a = w1_ref[0:D, :]
    wt_col = wt_ref[...]
    w1d_col = w1dt_ref[...]
    b1_col = b1_ref[...]
    b2_row = b2_ref[...]

    def row(b, carry):
        tlo = tlo_ref[pl.ds(b, 1), :]
        dlt = dlt_ref[pl.ds(b, 1), :]
        g = g_ref[pl.ds(b, 1), :]
        acc = acc_ref[pl.ds(b, 1), :]

        koh_t = (kk_k == g).astype(f32)
        lab_t = jnp.bitwise_and(jnp.right_shift(acc, kk_k), 1).astype(f32)
        lab = lax.dot_general(lab_t, eye_k, dn0, precision=hi,
                              preferred_element_type=f32)

        feat_t = lax.dot_general(emb_ref[...], koh_t, dn0,
                                 preferred_element_type=f32) + wt_col * tlo
        h_t = jnp.maximum(
            lax.dot_general(w1a, feat_t, dn0, preferred_element_type=f32)
            + w1d_col * dlt + b1_col, 0.0)
        logits = lax.dot_general(h_t, w2_ref[...], dn0,
                                 preferred_element_type=f32) + b2_row
        probs_ref[pl.ds(b, 1), :, :] = jax.nn.sigmoid(logits).reshape(1, S, K)
        label_ref[pl.ds(b, 1), :, :] = lab.reshape(1, S, K)
        return carry

    lax.fori_loop(0, B, row, jnp.int32(0))


def kernel(time_seqs, type_seqs, seq_non_pad_mask, uniform_rand, type_emb,
           w_time, W1, b1, W2, b2):
    del seq_non_pad_mask
    B, L = time_seqs.shape
    S = uniform_rand.shape[1]
    K, D = type_emb.shape
    f32 = jnp.float32

    first = time_seqs[:, 0]
    upper = jnp.maximum(time_seqs[:, -1] - _TAU, first)
    fs = jnp.concatenate(
        [jnp.broadcast_to(first[:, None], (B, _LANES)),
         jnp.broadcast_to((upper - first)[:, None], (B, _LANES))], axis=1)

    sc_fn = _make_sc_fn(B, L, S)
    tlo, dlt, g, acc = sc_fn(
        time_seqs, type_seqs.astype(jnp.int32), uniform_rand, fs)

    wt_col = w_time.reshape(D, 1)
    w1d_col = W1[D].reshape(D, 1)
    b1_col = b1.reshape(D, 1)
    b2_row = b2.reshape(1, K)

    probs, label = pl.pallas_call(
        _tc_body,
        out_shape=[
            jax.ShapeDtypeStruct((B, S, K), f32),
            jax.ShapeDtypeStruct((B, S, K), f32),
        ],
    )(tlo, dlt, g, acc, type_emb, wt_col, W1, w1d_col, b1_col, W2, b2_row)
    return (probs, label)


# speedup vs baseline: 1.3559x; 1.3559x over previous
import functools

import jax
import jax.numpy as jnp
from jax import lax
from jax.experimental import pallas as pl
from jax.experimental.pallas import tpu as pltpu
from jax.experimental.pallas import tpu_sc as plsc

_TAU = 10.0
_LANES = 16


def _make_sc_fn(B, L, S):
    f32, i32 = jnp.float32, jnp.int32
    NC = 2
    mesh = plsc.VectorSubcoreMesh(
        core_axis_name="c", subcore_axis_name="s", num_cores=NC, num_subcores=16)

    @functools.partial(
        pl.kernel,
        mesh=mesh,
        compiler_params=pltpu.CompilerParams(needs_layout_passes=False),
        out_type=[
            jax.ShapeDtypeStruct((B, S, 128), f32),
        ],
        scratch_types=[
            pltpu.VMEM((L,), f32),
            pltpu.VMEM((L,), i32),
            pltpu.VMEM((S,), f32),
            pltpu.VMEM((2 * _LANES,), f32),
            pltpu.VMEM((S, 128), f32),
        ],
    )
    def sc_fn(time_hbm, type_hbm, u_hbm, fs_hbm, aux_hbm,
              t_v, ty_v, u_v, fs_v, aux_v):
        wid = lax.axis_index("s") * NC + lax.axis_index("c")
        pltpu.sync_copy(time_hbm.at[wid], t_v)
        pltpu.sync_copy(type_hbm.at[wid], ty_v)
        pltpu.sync_copy(u_hbm.at[wid], u_v)
        pltpu.sync_copy(fs_hbm.at[wid], fs_v)

        first = fs_v[pl.ds(0, _LANES)]
        scale = fs_v[pl.ds(_LANES, _LANES)]

        def search(base):
            st = u_v[pl.ds(base, _LANES)] * scale + first
            sthi = st + _TAU
            pos = jnp.zeros((_LANES,), i32)
            pos2 = jnp.zeros((_LANES,), i32)
            step = L // 2
            while step >= 1:
                cand = pos + step
                cand2 = pos2 + step
                tc = plsc.load_gather(t_v, [cand])
                tc2 = plsc.load_gather(t_v, [cand2])
                pos = jnp.where(tc <= st, cand, pos)
                pos2 = jnp.where(tc2 <= sthi, cand2, pos2)
                step //= 2
            t_lo = plsc.load_gather(t_v, [pos])
            g16 = plsc.load_gather(ty_v, [pos])
            return st, pos, pos2, t_lo, g16

        def scan_store(base, srch):
            st, pos, pos2, t_lo, g16 = srch

            w = pos2 - pos

            def wbody(state):
                j, acc = state
                for r in range(4):
                    jr = j + r
                    idx = jnp.minimum(pos + 1 + jr, L - 1)
                    tyj = plsc.load_gather(ty_v, [idx])
                    bit = jnp.where(jr < w, jnp.left_shift(jnp.int32(1), tyj), 0)
                    acc = acc | bit
                return (j + jnp.int32(4), acc)

            _, acc = lax.while_loop(
                lambda s: jnp.any(s[0] < w), wbody,
                (jnp.int32(0), jnp.zeros((_LANES,), i32)))

            lane = lax.iota(i32, _LANES)
            srow = base + lane
            plsc.store_scatter(aux_v, [srow, jnp.zeros((_LANES,), i32)], t_lo)
            plsc.store_scatter(aux_v, [srow, jnp.full((_LANES,), 1, i32)], st - t_lo)
            plsc.store_scatter(aux_v, [srow, jnp.full((_LANES,), 2, i32)], g16.astype(f32))
            plsc.store_scatter(aux_v, [srow, jnp.full((_LANES,), 3, i32)], plsc.bitcast(acc, f32))

        def chunk(i, carry):
            base_a = i * 2 * _LANES
            base_b = base_a + _LANES
            sa = search(base_a)
            sb = search(base_b)
            scan_store(base_a, sa)
            scan_store(base_b, sb)
            return carry

        lax.fori_loop(0, S // (2 * _LANES), chunk, jnp.int32(0))

        pltpu.sync_copy(aux_v, aux_hbm.at[wid])

    return sc_fn


def _tc_body(aux_ref, emb_ref, wt_ref, w1_ref,
             b1_ref, w2_ref, b2_ref, probs_ref, label_ref):
    R = aux_ref.shape[0]
    K, D = emb_ref.shape
    f32, i32 = jnp.float32, jnp.int32
    tlo = aux_ref[:, 0:1]
    dlt = aux_ref[:, 1:2]
    g_col = aux_ref[:, 2:3].astype(i32)
    acc = lax.bitcast_convert_type(aux_ref[:, 3:4], i32)

    kk = lax.broadcasted_iota(i32, (R, K), 1)
    label_ref[...] = jnp.bitwise_and(jnp.right_shift(acc, kk), 1).astype(f32)
    koh = (kk == g_col).astype(f32)

    w1a = w1_ref[0:D, :]
    w1d = w1_ref[D:D + 1, :]
    feat = jnp.dot(koh, emb_ref[...], preferred_element_type=f32) + tlo * wt_ref[...]
    h = jnp.maximum(
        jnp.dot(feat, w1a, preferred_element_type=f32)
        + dlt * w1d + b1_ref[...], 0.0)
    logits = jnp.dot(h, w2_ref[...], preferred_element_type=f32) + b2_ref[...]
    probs_ref[...] = jax.nn.sigmoid(logits)


def kernel(time_seqs, type_seqs, seq_non_pad_mask, uniform_rand, type_emb,
           w_time, W1, b1, W2, b2):
    del seq_non_pad_mask
    B, L = time_seqs.shape
    S = uniform_rand.shape[1]
    K, D = type_emb.shape
    f32 = jnp.float32

    first = time_seqs[:, 0]
    upper = jnp.maximum(time_seqs[:, -1] - _TAU, first)
    fs = jnp.concatenate(
        [jnp.broadcast_to(first[:, None], (B, _LANES)),
         jnp.broadcast_to((upper - first)[:, None], (B, _LANES))], axis=1)

    sc_fn = _make_sc_fn(B, L, S)
    (aux,) = sc_fn(time_seqs, type_seqs.astype(jnp.int32), uniform_rand, fs)
    aux2 = aux.reshape(B * S, 128)

    wt = w_time.reshape(1, D)
    b1r = b1.reshape(1, D)
    b2r = b2.reshape(1, K)

    probs, label = pl.pallas_call(
        _tc_body,
        out_shape=[
            jax.ShapeDtypeStruct((B * S, K), f32),
            jax.ShapeDtypeStruct((B * S, K), f32),
        ],
    )(aux2, type_emb, wt, W1, b1r, W2, b2r)
    return (probs.reshape(B, S, K), label.reshape(B, S, K))


# speedup vs baseline: 1.3809x; 1.0185x over previous
import functools

import jax
import jax.numpy as jnp
from jax import lax
from jax.experimental import pallas as pl
from jax.experimental.pallas import tpu as pltpu
from jax.experimental.pallas import tpu_sc as plsc

_TAU = 10.0
_LANES = 16


def _make_sc_fn(B, L, S):
    f32, i32 = jnp.float32, jnp.int32
    NC = 2
    mesh = plsc.VectorSubcoreMesh(
        core_axis_name="c", subcore_axis_name="s", num_cores=NC, num_subcores=16)

    @functools.partial(
        pl.kernel,
        mesh=mesh,
        compiler_params=pltpu.CompilerParams(needs_layout_passes=False),
        out_type=[
            jax.ShapeDtypeStruct((B, S, 128), f32),
        ],
        scratch_types=[
            pltpu.VMEM((L,), f32),
            pltpu.VMEM((L,), i32),
            pltpu.VMEM((S,), f32),
            pltpu.VMEM((2 * _LANES,), f32),
            pltpu.VMEM((S, 128), f32),
        ],
    )
    def sc_fn(time_hbm, type_hbm, u_hbm, fs_hbm, aux_hbm,
              t_v, ty_v, u_v, fs_v, aux_v):
        wid = lax.axis_index("s") * NC + lax.axis_index("c")
        pltpu.sync_copy(time_hbm.at[wid], t_v)
        pltpu.sync_copy(type_hbm.at[wid], ty_v)
        pltpu.sync_copy(u_hbm.at[wid], u_v)
        pltpu.sync_copy(fs_hbm.at[wid], fs_v)

        first = fs_v[pl.ds(0, _LANES)]
        scale = fs_v[pl.ds(_LANES, _LANES)]

        def search(base):
            st = u_v[pl.ds(base, _LANES)] * scale + first
            sthi = st + _TAU
            pos = jnp.zeros((_LANES,), i32)
            pos2 = jnp.zeros((_LANES,), i32)
            step = L // 2
            while step >= 1:
                cand = pos + step
                cand2 = pos2 + step
                tc = plsc.load_gather(t_v, [cand])
                tc2 = plsc.load_gather(t_v, [cand2])
                pos = jnp.where(tc <= st, cand, pos)
                pos2 = jnp.where(tc2 <= sthi, cand2, pos2)
                step //= 2
            t_lo = plsc.load_gather(t_v, [pos])
            g16 = plsc.load_gather(ty_v, [pos])
            return st, pos, pos2, t_lo, g16

        def scan_store(base, srch):
            st, pos, pos2, t_lo, g16 = srch

            w = pos2 - pos

            def wbody(state):
                j, acc = state
                for r in range(4):
                    jr = j + r
                    idx = jnp.minimum(pos + 1 + jr, L - 1)
                    tyj = plsc.load_gather(ty_v, [idx])
                    bit = jnp.where(jr < w, jnp.left_shift(jnp.int32(1), tyj), 0)
                    acc = acc | bit
                return (j + jnp.int32(4), acc)

            _, acc = lax.while_loop(
                lambda s: jnp.any(s[0] < w), wbody,
                (jnp.int32(0), jnp.zeros((_LANES,), i32)))

            lane = lax.iota(i32, _LANES)
            srow = base + lane
            plsc.store_scatter(aux_v, [srow, jnp.zeros((_LANES,), i32)], t_lo)
            plsc.store_scatter(aux_v, [srow, jnp.full((_LANES,), 1, i32)], st - t_lo)
            plsc.store_scatter(aux_v, [srow, jnp.full((_LANES,), 2, i32)], g16.astype(f32))
            plsc.store_scatter(aux_v, [srow, jnp.full((_LANES,), 3, i32)], plsc.bitcast(acc, f32))

        def chunk(i, carry):
            base_a = i * 2 * _LANES
            base_b = base_a + _LANES
            sa = search(base_a)
            sb = search(base_b)
            scan_store(base_a, sa)
            scan_store(base_b, sb)
            return carry

        lax.fori_loop(0, S // (2 * _LANES), chunk, jnp.int32(0))

        pltpu.sync_copy(aux_v, aux_hbm.at[wid])

    return sc_fn


def _tc_body(aux_ref, emb_ref, wt_ref, w1_ref,
             b1_ref, w2_ref, b2_ref, probs_ref, label_ref):
    R = aux_ref.shape[0]
    K, D = emb_ref.shape
    f32, i32 = jnp.float32, jnp.int32
    bb, ss, _ = probs_ref.shape
    tlo = aux_ref[:, 0:1]
    dlt = aux_ref[:, 1:2]
    g_col = aux_ref[:, 2:3].astype(i32)
    acc = lax.bitcast_convert_type(aux_ref[:, 3:4], i32)

    kk = lax.broadcasted_iota(i32, (R, K), 1)
    label = jnp.bitwise_and(jnp.right_shift(acc, kk), 1).astype(f32)
    koh = (kk == g_col).astype(f32)

    w1a = w1_ref[0:D, :]
    w1d = w1_ref[D:D + 1, :]
    feat = jnp.dot(koh, emb_ref[...], preferred_element_type=f32) + tlo * wt_ref[...]
    h = jnp.maximum(
        jnp.dot(feat, w1a, preferred_element_type=f32)
        + dlt * w1d + b1_ref[...], 0.0)
    logits = jnp.dot(h, w2_ref[...], preferred_element_type=f32) + b2_ref[...]
    probs_ref[...] = jax.nn.sigmoid(logits).reshape(bb, ss, K)
    label_ref[...] = label.reshape(bb, ss, K)


def kernel(time_seqs, type_seqs, seq_non_pad_mask, uniform_rand, type_emb,
           w_time, W1, b1, W2, b2):
    del seq_non_pad_mask
    B, L = time_seqs.shape
    S = uniform_rand.shape[1]
    K, D = type_emb.shape
    f32 = jnp.float32

    first = time_seqs[:, 0]
    upper = jnp.maximum(time_seqs[:, -1] - _TAU, first)
    fs = jnp.concatenate(
        [jnp.broadcast_to(first[:, None], (B, _LANES)),
         jnp.broadcast_to((upper - first)[:, None], (B, _LANES))], axis=1)

    sc_fn = _make_sc_fn(B, L, S)
    (aux,) = sc_fn(time_seqs, type_seqs.astype(jnp.int32), uniform_rand, fs)
    aux2 = aux.reshape(B * S, 128)

    wt = w_time.reshape(1, D)
    b1r = b1.reshape(1, D)
    b2r = b2.reshape(1, K)

    GB = 8
    probs, label = pl.pallas_call(
        _tc_body,
        grid=(B // GB,),
        in_specs=[
            pl.BlockSpec((GB * S, 128), lambda i: (i, 0)),
            pl.BlockSpec((K, D), lambda i: (0, 0)),
            pl.BlockSpec((1, D), lambda i: (0, 0)),
            pl.BlockSpec((D + 1, D), lambda i: (0, 0)),
            pl.BlockSpec((1, D), lambda i: (0, 0)),
            pl.BlockSpec((D, K), lambda i: (0, 0)),
            pl.BlockSpec((1, K), lambda i: (0, 0)),
        ],
        out_specs=[
            pl.BlockSpec((GB, S, K), lambda i: (i, 0, 0)),
            pl.BlockSpec((GB, S, K), lambda i: (i, 0, 0)),
        ],
        out_shape=[
            jax.ShapeDtypeStruct((B, S, K), f32),
            jax.ShapeDtypeStruct((B, S, K), f32),
        ],
    )(aux2, type_emb, wt, W1, b1r, W2, b2r)
    return (probs, label)


# speedup vs baseline: 1.4151x; 1.0248x over previous
import functools

import jax
import jax.numpy as jnp
from jax import lax
from jax.experimental import pallas as pl
from jax.experimental.pallas import tpu as pltpu
from jax.experimental.pallas import tpu_sc as plsc

_TAU = 10.0
_LANES = 16


def _make_sc_fn(B, L, S):
    f32, i32 = jnp.float32, jnp.int32
    NC = 2
    mesh = plsc.VectorSubcoreMesh(
        core_axis_name="c", subcore_axis_name="s", num_cores=NC, num_subcores=16)

    @functools.partial(
        pl.kernel,
        mesh=mesh,
        compiler_params=pltpu.CompilerParams(needs_layout_passes=False),
        out_type=[
            jax.ShapeDtypeStruct((B, S, 128), f32),
        ],
        scratch_types=[
            pltpu.VMEM((L,), f32),
            pltpu.VMEM((L,), i32),
            pltpu.VMEM((S,), f32),
            pltpu.VMEM((S, 128), f32),
            pltpu.SemaphoreType.DMA,
        ],
    )
    def sc_fn(time_hbm, type_hbm, u_hbm, aux_hbm,
              t_v, ty_v, u_v, aux_v, sem):
        wid = lax.axis_index("s") * NC + lax.axis_index("c")
        pltpu.sync_copy(time_hbm.at[wid], t_v)
        pltpu.sync_copy(type_hbm.at[wid], ty_v)
        pltpu.sync_copy(u_hbm.at[wid], u_v)

        first = -plsc.cummax(-t_v[pl.ds(0, _LANES)])
        final = plsc.cummax(lax.rev(t_v[pl.ds(L - _LANES, _LANES)], (0,)))
        scale = jnp.maximum(final - _TAU, first) - first

        def search(base):
            st = u_v[pl.ds(base, _LANES)] * scale + first
            sthi = st + _TAU
            pos = jnp.zeros((_LANES,), i32)
            pos2 = jnp.zeros((_LANES,), i32)
            step = L // 2
            while step >= 1:
                cand = pos + step
                cand2 = pos2 + step
                tc = plsc.load_gather(t_v, [cand])
                tc2 = plsc.load_gather(t_v, [cand2])
                pos = jnp.where(tc <= st, cand, pos)
                pos2 = jnp.where(tc2 <= sthi, cand2, pos2)
                step //= 2
            t_lo = plsc.load_gather(t_v, [pos])
            g16 = plsc.load_gather(ty_v, [pos])
            return st, pos, pos2, t_lo, g16

        def scan_store(base, srch):
            st, pos, pos2, t_lo, g16 = srch

            w = pos2 - pos

            def wbody(state):
                j, acc = state
                for r in range(4):
                    jr = j + r
                    idx = jnp.minimum(pos + 1 + jr, L - 1)
                    tyj = plsc.load_gather(ty_v, [idx])
                    bit = jnp.where(jr < w, jnp.left_shift(jnp.int32(1), tyj), 0)
                    acc = acc | bit
                return (j + jnp.int32(4), acc)

            _, acc = lax.while_loop(
                lambda s: jnp.any(s[0] < w), wbody,
                (jnp.int32(0), jnp.zeros((_LANES,), i32)))

            lane = lax.iota(i32, _LANES)
            srow = base + lane
            plsc.store_scatter(aux_v, [srow, jnp.zeros((_LANES,), i32)], t_lo)
            plsc.store_scatter(aux_v, [srow, jnp.full((_LANES,), 1, i32)], st - t_lo)
            plsc.store_scatter(aux_v, [srow, jnp.full((_LANES,), 2, i32)], g16.astype(f32))
            plsc.store_scatter(aux_v, [srow, jnp.full((_LANES,), 3, i32)], plsc.bitcast(acc, f32))

        def chunk(i, carry):
            base_a = i * 2 * _LANES
            base_b = base_a + _LANES
            sa = search(base_a)
            sb = search(base_b)
            scan_store(base_a, sa)
            scan_store(base_b, sb)
            return carry

        half = S // 2
        lax.fori_loop(0, S // (4 * _LANES), chunk, jnp.int32(0))
        cp1 = pltpu.make_async_copy(
            aux_v.at[pl.ds(0, half)], aux_hbm.at[wid, pl.ds(0, half)], sem)
        cp1.start()
        lax.fori_loop(S // (4 * _LANES), S // (2 * _LANES), chunk, jnp.int32(0))
        cp2 = pltpu.make_async_copy(
            aux_v.at[pl.ds(half, half)], aux_hbm.at[wid, pl.ds(half, half)], sem)
        cp2.start()
        cp1.wait()
        cp2.wait()

    return sc_fn


def _tc_body(aux_ref, emb_ref, wt_ref, w1_ref,
             b1_ref, w2_ref, b2_ref, probs_ref, label_ref):
    R = aux_ref.shape[0]
    K, D = emb_ref.shape
    f32, i32 = jnp.float32, jnp.int32
    bb, ss, _ = probs_ref.shape
    tlo = aux_ref[:, 0:1]
    dlt = aux_ref[:, 1:2]
    g_col = aux_ref[:, 2:3].astype(i32)
    acc = lax.bitcast_convert_type(aux_ref[:, 3:4], i32)

    kk = lax.broadcasted_iota(i32, (R, K), 1)
    label = jnp.bitwise_and(jnp.right_shift(acc, kk), 1).astype(f32)
    koh = (kk == g_col).astype(f32)

    w1a = w1_ref[0:D, :]
    w1d = w1_ref[D:D + 1, :]
    feat = jnp.dot(koh, emb_ref[...], preferred_element_type=f32) + tlo * wt_ref[...]
    h = jnp.maximum(
        jnp.dot(feat, w1a, preferred_element_type=f32)
        + dlt * w1d + b1_ref[...], 0.0)
    logits = jnp.dot(h, w2_ref[...], preferred_element_type=f32) + b2_ref[...]
    probs_ref[...] = jax.nn.sigmoid(logits).reshape(bb, ss, K)
    label_ref[...] = label.reshape(bb, ss, K)


def kernel(time_seqs, type_seqs, seq_non_pad_mask, uniform_rand, type_emb,
           w_time, W1, b1, W2, b2):
    del seq_non_pad_mask
    B, L = time_seqs.shape
    S = uniform_rand.shape[1]
    K, D = type_emb.shape
    f32 = jnp.float32

    sc_fn = _make_sc_fn(B, L, S)
    (aux,) = sc_fn(time_seqs, type_seqs.astype(jnp.int32), uniform_rand)
    aux2 = aux.reshape(B * S, 128)

    wt = w_time.reshape(1, D)
    b1r = b1.reshape(1, D)
    b2r = b2.reshape(1, K)

    GB = 8
    probs, label = pl.pallas_call(
        _tc_body,
        grid=(B // GB,),
        in_specs=[
            pl.BlockSpec((GB * S, 128), lambda i: (i, 0)),
            pl.BlockSpec((K, D), lambda i: (0, 0)),
            pl.BlockSpec((1, D), lambda i: (0, 0)),
            pl.BlockSpec((D + 1, D), lambda i: (0, 0)),
            pl.BlockSpec((1, D), lambda i: (0, 0)),
            pl.BlockSpec((D, K), lambda i: (0, 0)),
            pl.BlockSpec((1, K), lambda i: (0, 0)),
        ],
        out_specs=[
            pl.BlockSpec((GB, S, K), lambda i: (i, 0, 0)),
            pl.BlockSpec((GB, S, K), lambda i: (i, 0, 0)),
        ],
        out_shape=[
            jax.ShapeDtypeStruct((B, S, K), f32),
            jax.ShapeDtypeStruct((B, S, K), f32),
        ],
    )(aux2, type_emb, wt, W1, b1r, W2, b2r)
    return (probs, label)
